# baseline (speedup 1.0000x reference)
import functools

import jax
import jax.numpy as jnp
from jax import lax
from jax.experimental import pallas as pl
from jax.experimental.pallas import tpu as pltpu
from jax.experimental.pallas import tpu_sc as plsc

_CHUNK = 128
_LANES = 16
_NWORK = 32
_NSLOT = 256


def _sc_main(v, slen, cums, words, table_pad):
    b_dim, t_dim, d_dim = v.shape
    max_my = _NSLOT // _NWORK
    nd = d_dim // _LANES
    mesh = plsc.VectorSubcoreMesh(core_axis_name="c", subcore_axis_name="s")

    @functools.partial(
        pl.kernel,
        out_type=jax.ShapeDtypeStruct((_NSLOT, 4, d_dim), jnp.float32),
        mesh=mesh,
        scratch_types=[
            pltpu.VMEM((_LANES,), jnp.int32),
            pltpu.VMEM((_LANES,), jnp.int32),
            pltpu.VMEM((max_my, _CHUNK), jnp.int32),
            pltpu.VMEM((max_my, _CHUNK), jnp.float32),
            pltpu.VMEM((2, _CHUNK, d_dim), jnp.float32),
            pltpu.VMEM((max_my, 4, d_dim), jnp.float32),
            pltpu.SemaphoreType.DMA,
            pltpu.SemaphoreType.DMA((max_my,)),
            pltpu.SemaphoreType.DMA((2,)),
            pltpu.SemaphoreType.DMA,
        ],
        compiler_params=pltpu.CompilerParams(needs_layout_passes=False),
    )
    def main_kernel(
        v_hbm, slen_hbm, cums_hbm, words_hbm, table_hbm, out_hbm,
        len_v, cum_v, idx_v, w_v, vbuf, pbuf, semi, semw, semv, semo,
    ):
        wid = lax.axis_index("s") * 2 + lax.axis_index("c")

        pltpu.sync_copy(slen_hbm, len_v)
        pltpu.sync_copy(cums_hbm, cum_v)
        lvec = len_v[...]
        cumv = cum_v[...]
        rows16 = lax.iota(jnp.int32, _LANES)
        ncv = (lvec + (_CHUNK - 1)) // _CHUNK
        cumex = cumv - ncv
        nc_total = jnp.max(cumv)

        def chunk_info(q):
            row = jnp.minimum(
                jnp.sum(jnp.where(cumv <= q, 1, 0)), _LANES - 1
            )
            onerow = rows16 == row
            lr = jnp.max(jnp.where(onerow, lvec, 0))
            base = jnp.max(jnp.where(onerow, cumex, 0))
            return row, q - base, lr

        @pl.when(wid < nc_total)
        def _():
            row0, c0, _ = chunk_info(wid)
            pltpu.async_copy(
                v_hbm.at[row0, pl.ds(c0 * _CHUNK, _CHUNK)],
                vbuf.at[0],
                semv.at[0],
            )

        for t in range(max_my):
            q = wid + _NWORK * t

            @pl.when(q < nc_total)
            def _():
                row, c, _ = chunk_info(q)
                pltpu.async_copy(
                    words_hbm.at[row, pl.ds(c * _CHUNK, _CHUNK)],
                    idx_v.at[t],
                    semi,
                )
        for t in range(max_my):
            q = wid + _NWORK * t

            @pl.when(q < nc_total)
            def _():
                row, c, _ = chunk_info(q)
                pltpu.make_async_copy(
                    words_hbm.at[row, pl.ds(c * _CHUNK, _CHUNK)],
                    idx_v.at[t],
                    semi,
                ).wait()
        for t in range(max_my):
            q = wid + _NWORK * t

            @pl.when(q < nc_total)
            def _():
                pltpu.async_copy(
                    table_hbm.at[idx_v.at[t]], w_v.at[t], semw.at[t]
                )

        zvec = jnp.zeros((_LANES,), jnp.float32)

        for t in range(max_my):
            for l in range(nd):
                pbuf[t, 3, pl.ds(l * _LANES, _LANES)] = zvec

        def chunk_body(t):
            q = wid + _NWORK * t
            buf = t % 2
            valid = q < nc_total

            @pl.when(wid + _NWORK * (t + 1) < nc_total)
            def _():
                rown, cn, _ = chunk_info(wid + _NWORK * (t + 1))
                pltpu.async_copy(
                    v_hbm.at[rown, pl.ds(cn * _CHUNK, _CHUNK)],
                    vbuf.at[(t + 1) % 2],
                    semv.at[(t + 1) % 2],
                )

            @pl.when(valid)
            def _():
                row, c, lr = chunk_info(q)
                pltpu.make_async_copy(
                    table_hbm.at[idx_v.at[t]], w_v.at[t], semw.at[t]
                ).wait()
                rem = lr - (lr // _CHUNK) * _CHUNK
                partial = (rem > 0) & (c == lr // _CHUNK)

                @pl.when(partial)
                def _():
                    lanes = lax.iota(jnp.int32, _LANES)
                    for g in range(_CHUNK // _LANES):
                        pos = g * _LANES + lanes
                        wrow = w_v[t, pl.ds(g * _LANES, _LANES)]
                        w_v[t, pl.ds(g * _LANES, _LANES)] = jnp.where(
                            pos < rem, wrow, 0.0
                        )

                pltpu.make_async_copy(
                    v_hbm.at[row, pl.ds(c * _CHUNK, _CHUNK)],
                    vbuf.at[buf],
                    semv.at[buf],
                ).wait()

                @pl.when(partial)
                def _():
                    def zrow(rowi, cz):
                        for l in range(nd):
                            vbuf[buf, rowi, pl.ds(l * _LANES, _LANES)] = zvec
                        return cz

                    lax.fori_loop(rem, _CHUNK, zrow, 0, unroll=False)

                zeros = tuple(
                    jnp.zeros((_LANES,), jnp.float32) for _ in range(3 * nd)
                )

                def group_body(g, acc_g):
                    wv = w_v[t, pl.ds(g * _LANES, _LANES)]
                    lanes = lax.iota(jnp.int32, _LANES)
                    accs = list(acc_g)
                    for k in range(_LANES):
                        wk = jnp.sum(jnp.where(lanes == k, wv, 0.0))
                        bw = jnp.broadcast_to(wk, (_LANES,))
                        tok = g * _LANES + k
                        for l in range(nd):
                            vt = vbuf[buf, tok, pl.ds(l * _LANES, _LANES)]
                            accs[l] = accs[l] + vt
                            accs[nd + l] = accs[nd + l] + jnp.abs(vt)
                            accs[2 * nd + l] = accs[2 * nd + l] + bw * vt
                    return tuple(accs)

                acc = lax.fori_loop(
                    0, _CHUNK // _LANES, group_body, zeros, unroll=False
                )
                for a in range(3):
                    for l in range(nd):
                        pbuf[t, a, pl.ds(l * _LANES, _LANES)] = acc[
                            a * nd + l
                        ]

            @pl.when(jnp.logical_not(valid))
            def _():
                for a in range(3):
                    for l in range(nd):
                        pbuf[t, a, pl.ds(l * _LANES, _LANES)] = zvec

        for t in range(max_my):
            chunk_body(t)

        for t in range(max_my):
            pltpu.async_copy(pbuf.at[t], out_hbm.at[wid + _NWORK * t], semo)
        for t in range(max_my):
            pltpu.make_async_copy(
                pbuf.at[t], out_hbm.at[wid + _NWORK * t], semo
            ).wait()

    return main_kernel(v, slen, cums, words, table_pad)


def _tc_combine(partials, slen, cums):
    nslot, four_d = partials.shape
    b_dim = slen.shape[0]
    d_dim = four_d // 4

    def body(p_ref, slen_ref, cums_ref, y_ref, yh_ref):
        cum_incl = cums_ref[...]
        ncv = (slen_ref[...] + (_CHUNK - 1)) // _CHUNK
        cum_excl = cum_incl - ncv
        q = lax.broadcasted_iota(jnp.int32, (b_dim, nslot), 1)
        own = (cum_excl[:, None] <= q) & (q < cum_incl[:, None])
        s = jnp.dot(
            own.astype(jnp.float32),
            p_ref[...],
            preferred_element_type=jnp.float32,
        )
        y_ref[...] = s[:, :d_dim] / s[:, d_dim : 2 * d_dim]
        yh_ref[...] = s[:, 2 * d_dim : 3 * d_dim]

    return pl.pallas_call(
        body,
        out_shape=[
            jax.ShapeDtypeStruct((b_dim, d_dim), jnp.float32),
            jax.ShapeDtypeStruct((b_dim, d_dim), jnp.float32),
        ],
    )(partials, slen, cums)


def kernel(vector_sequence, sentence_length, word_sequence, W):
    b_dim, t_dim, d_dim = vector_sequence.shape
    vocab = W.shape[0]
    slen = sentence_length.astype(jnp.int32)
    words = word_sequence.astype(jnp.int32)
    cums = jnp.cumsum((slen + (_CHUNK - 1)) // _CHUNK).astype(jnp.int32)
    vpad = ((vocab + 1023) // 1024) * 1024
    table_pad = jnp.pad(W.astype(jnp.float32), (0, vpad - vocab))
    partials = _sc_main(vector_sequence, slen, cums, words, table_pad)
    y, y_hat = _tc_combine(
        partials.reshape(_NSLOT, 4 * d_dim), slen, cums
    )
    return (y, y_hat)


# speedup vs baseline: 1.2083x; 1.2083x over previous
import functools

import jax
import jax.numpy as jnp
from jax import lax
from jax.experimental import pallas as pl
from jax.experimental.pallas import tpu as pltpu
from jax.experimental.pallas import tpu_sc as plsc

_CHUNK = 128
_LANES = 16
_NWORK = 32
_NSLOT = 256


def _sc_main(v, slen, cums, words, table_pad):
    b_dim, t_dim, d_dim = v.shape
    max_my = _NSLOT // _NWORK
    nd = d_dim // _LANES
    mesh = plsc.VectorSubcoreMesh(core_axis_name="c", subcore_axis_name="s")

    @functools.partial(
        pl.kernel,
        out_type=jax.ShapeDtypeStruct((_NSLOT, 3, d_dim), jnp.float32),
        mesh=mesh,
        scratch_types=[
            pltpu.VMEM((_LANES,), jnp.int32),
            pltpu.VMEM((_LANES,), jnp.int32),
            pltpu.VMEM((max_my, _CHUNK), jnp.int32),
            pltpu.VMEM((max_my, _CHUNK), jnp.float32),
            pltpu.VMEM((2, _CHUNK, d_dim), jnp.float32),
            pltpu.VMEM((3, d_dim), jnp.float32),
            pltpu.SemaphoreType.DMA,
            pltpu.SemaphoreType.DMA((2,)),
            pltpu.SemaphoreType.DMA,
        ],
        compiler_params=pltpu.CompilerParams(needs_layout_passes=False),
    )
    def main_kernel(
        v_hbm, slen_hbm, cums_hbm, words_hbm, table_hbm, out_hbm,
        len_v, cum_v, idx_v, w_v, vbuf, pbuf, semw, semv, semo,
    ):
        wid = lax.axis_index("s") * 2 + lax.axis_index("c")

        pltpu.sync_copy(slen_hbm, len_v)
        pltpu.sync_copy(cums_hbm, cum_v)
        lvec = len_v[...]
        cumv = cum_v[...]
        rows16 = lax.iota(jnp.int32, _LANES)
        ncv = (lvec + (_CHUNK - 1)) // _CHUNK
        cumex = cumv - ncv
        nc_total = jnp.max(cumv)

        def chunk_info(q):
            row = jnp.minimum(
                jnp.sum(jnp.where(cumv <= q, 1, 0)), _LANES - 1
            )
            onerow = rows16 == row
            lr = jnp.max(jnp.where(onerow, lvec, 0))
            base = jnp.max(jnp.where(onerow, cumex, 0))
            return row, q - base, lr

        @pl.when(wid < nc_total)
        def _():
            row0, c0, _ = chunk_info(wid)
            pltpu.async_copy(
                v_hbm.at[row0, pl.ds(c0 * _CHUNK, _CHUNK)],
                vbuf.at[0],
                semv.at[0],
            )

        for t in range(max_my):
            q = wid + _NWORK * t

            @pl.when(q < nc_total)
            def _():
                row, c, _ = chunk_info(q)
                pltpu.async_copy(
                    words_hbm.at[row, pl.ds(c * _CHUNK, _CHUNK)],
                    idx_v.at[t],
                    semw,
                )
        for t in range(max_my):
            q = wid + _NWORK * t

            @pl.when(q < nc_total)
            def _():
                row, c, _ = chunk_info(q)
                pltpu.make_async_copy(
                    words_hbm.at[row, pl.ds(c * _CHUNK, _CHUNK)],
                    idx_v.at[t],
                    semw,
                ).wait()
        for t in range(max_my):
            q = wid + _NWORK * t

            @pl.when(q < nc_total)
            def _():
                pltpu.async_copy(table_hbm.at[idx_v.at[t]], w_v.at[t], semw)
        for t in range(max_my):
            q = wid + _NWORK * t

            @pl.when(q < nc_total)
            def _():
                pltpu.make_async_copy(
                    table_hbm.at[idx_v.at[t]], w_v.at[t], semw
                ).wait()

        for t in range(max_my):
            q = wid + _NWORK * t

            @pl.when(q < nc_total)
            def _():
                _, c, lr = chunk_info(q)
                rem = lr - (lr // _CHUNK) * _CHUNK

                @pl.when((rem > 0) & (c == lr // _CHUNK))
                def _():
                    lanes = lax.iota(jnp.int32, _LANES)
                    for g in range(_CHUNK // _LANES):
                        pos = g * _LANES + lanes
                        wrow = w_v[t, pl.ds(g * _LANES, _LANES)]
                        w_v[t, pl.ds(g * _LANES, _LANES)] = jnp.where(
                            pos < rem, wrow, 0.0
                        )

        zvec = jnp.zeros((_LANES,), jnp.float32)

        def chunk_body(t, carry):
            q = wid + _NWORK * t
            buf = t % 2
            valid = q < nc_total

            @pl.when(wid + _NWORK * (t + 1) < nc_total)
            def _():
                rown, cn, _ = chunk_info(wid + _NWORK * (t + 1))
                pltpu.async_copy(
                    v_hbm.at[rown, pl.ds(cn * _CHUNK, _CHUNK)],
                    vbuf.at[(t + 1) % 2],
                    semv.at[(t + 1) % 2],
                )

            @pl.when(t > 0)
            def _():
                pltpu.make_async_copy(
                    pbuf, out_hbm.at[q - _NWORK], semo
                ).wait()

            @pl.when(valid)
            def _():
                row, c, lr = chunk_info(q)
                pltpu.make_async_copy(
                    v_hbm.at[row, pl.ds(c * _CHUNK, _CHUNK)],
                    vbuf.at[buf],
                    semv.at[buf],
                ).wait()
                rem = lr - (lr // _CHUNK) * _CHUNK

                @pl.when((rem > 0) & (c == lr // _CHUNK))
                def _():
                    def zrow(rowi, cz):
                        for l in range(nd):
                            vbuf[buf, rowi, pl.ds(l * _LANES, _LANES)] = zvec
                        return cz

                    lax.fori_loop(rem, _CHUNK, zrow, 0, unroll=False)

                zeros = tuple(
                    jnp.zeros((_LANES,), jnp.float32) for _ in range(3 * nd)
                )

                def group_body(g, acc_g):
                    wv = w_v[t, pl.ds(g * _LANES, _LANES)]
                    lanes = lax.iota(jnp.int32, _LANES)
                    accs = list(acc_g)
                    for k in range(_LANES):
                        wk = jnp.sum(jnp.where(lanes == k, wv, 0.0))
                        bw = jnp.broadcast_to(wk, (_LANES,))
                        tok = g * _LANES + k
                        for l in range(nd):
                            vt = vbuf[buf, tok, pl.ds(l * _LANES, _LANES)]
                            accs[l] = accs[l] + vt
                            accs[nd + l] = accs[nd + l] + jnp.abs(vt)
                            accs[2 * nd + l] = accs[2 * nd + l] + bw * vt
                    return tuple(accs)

                acc = lax.fori_loop(
                    0, _CHUNK // _LANES, group_body, zeros, unroll=False
                )
                for a in range(3):
                    for l in range(nd):
                        pbuf[a, pl.ds(l * _LANES, _LANES)] = acc[a * nd + l]

            @pl.when(jnp.logical_not(valid))
            def _():
                for a in range(3):
                    for l in range(nd):
                        pbuf[a, pl.ds(l * _LANES, _LANES)] = zvec

            pltpu.async_copy(pbuf, out_hbm.at[q], semo)
            return carry

        lax.fori_loop(0, max_my, chunk_body, 0, unroll=False)
        pltpu.make_async_copy(
            pbuf, out_hbm.at[wid + _NWORK * (max_my - 1)], semo
        ).wait()

    return main_kernel(v, slen, cums, words, table_pad)


def _tc_combine(partials, slen, cums):
    nslot, three_d = partials.shape
    b_dim = slen.shape[0]
    d_dim = three_d // 3

    def body(p_ref, slen_ref, cums_ref, y_ref, yh_ref):
        cum_incl = cums_ref[...]
        ncv = (slen_ref[...] + (_CHUNK - 1)) // _CHUNK
        cum_excl = cum_incl - ncv
        q = lax.broadcasted_iota(jnp.int32, (b_dim, nslot), 1)
        own = (cum_excl[:, None] <= q) & (q < cum_incl[:, None])
        s = jnp.dot(
            own.astype(jnp.float32),
            p_ref[...],
            preferred_element_type=jnp.float32,
        )
        y_ref[...] = s[:, :d_dim] / s[:, d_dim : 2 * d_dim]
        yh_ref[...] = s[:, 2 * d_dim :]

    return pl.pallas_call(
        body,
        out_shape=[
            jax.ShapeDtypeStruct((b_dim, d_dim), jnp.float32),
            jax.ShapeDtypeStruct((b_dim, d_dim), jnp.float32),
        ],
    )(partials, slen, cums)


def kernel(vector_sequence, sentence_length, word_sequence, W):
    b_dim, t_dim, d_dim = vector_sequence.shape
    vocab = W.shape[0]
    slen = sentence_length.astype(jnp.int32)
    words = word_sequence.astype(jnp.int32)
    cums = jnp.cumsum((slen + (_CHUNK - 1)) // _CHUNK).astype(jnp.int32)
    vpad = ((vocab + 1023) // 1024) * 1024
    table_pad = jnp.pad(W.astype(jnp.float32), (0, vpad - vocab))
    partials = _sc_main(vector_sequence, slen, cums, words, table_pad)
    y, y_hat = _tc_combine(
        partials.reshape(_NSLOT, 3 * d_dim), slen, cums
    )
    return (y, y_hat)


# speedup vs baseline: 1.2236x; 1.0127x over previous
import functools

import jax
import jax.numpy as jnp
from jax import lax
from jax.experimental import pallas as pl
from jax.experimental.pallas import tpu as pltpu
from jax.experimental.pallas import tpu_sc as plsc

_CHUNK = 128
_LANES = 16
_NWORK = 32
_NSLOT = 256


def _sc_main(v, slen, cums, words, table_pad):
    b_dim, t_dim, d_dim = v.shape
    max_my = _NSLOT // _NWORK
    nd = d_dim // _LANES
    mesh = plsc.VectorSubcoreMesh(core_axis_name="c", subcore_axis_name="s")

    @functools.partial(
        pl.kernel,
        out_type=jax.ShapeDtypeStruct((_NSLOT, 3, d_dim), jnp.float32),
        mesh=mesh,
        scratch_types=[
            pltpu.VMEM((_LANES,), jnp.int32),
            pltpu.VMEM((_LANES,), jnp.int32),
            pltpu.VMEM((max_my, _CHUNK), jnp.int32),
            pltpu.VMEM((max_my, _CHUNK), jnp.float32),
            pltpu.VMEM((2, _CHUNK, d_dim), jnp.float32),
            pltpu.VMEM((3, d_dim), jnp.float32),
            pltpu.SemaphoreType.DMA,
            pltpu.SemaphoreType.DMA((2,)),
            pltpu.SemaphoreType.DMA,
        ],
        compiler_params=pltpu.CompilerParams(needs_layout_passes=False),
    )
    def main_kernel(
        v_hbm, slen_hbm, cums_hbm, words_hbm, table_hbm, out_hbm,
        len_v, cum_v, idx_v, w_v, vbuf, pbuf, semw, semv, semo,
    ):
        wid = lax.axis_index("s") * 2 + lax.axis_index("c")

        pltpu.sync_copy(slen_hbm, len_v)
        pltpu.sync_copy(cums_hbm, cum_v)
        lvec = len_v[...]
        cumv = cum_v[...]
        rows16 = lax.iota(jnp.int32, _LANES)
        ncv = (lvec + (_CHUNK - 1)) // _CHUNK
        cumex = cumv - ncv
        nc_total = jnp.max(cumv)

        def chunk_info(q):
            row = jnp.minimum(
                jnp.sum(jnp.where(cumv <= q, 1, 0)), _LANES - 1
            )
            onerow = rows16 == row
            lr = jnp.max(jnp.where(onerow, lvec, 0))
            base = jnp.max(jnp.where(onerow, cumex, 0))
            return row, q - base, lr

        @pl.when(wid < nc_total)
        def _():
            row0, c0, _ = chunk_info(wid)
            pltpu.async_copy(
                v_hbm.at[row0, pl.ds(c0 * _CHUNK, _CHUNK)],
                vbuf.at[0],
                semv.at[0],
            )

        for t in range(max_my):
            q = wid + _NWORK * t

            @pl.when(q < nc_total)
            def _():
                row, c, _ = chunk_info(q)
                pltpu.async_copy(
                    words_hbm.at[row, pl.ds(c * _CHUNK, _CHUNK)],
                    idx_v.at[t],
                    semw,
                )
        for t in range(max_my):
            q = wid + _NWORK * t

            @pl.when(q < nc_total)
            def _():
                row, c, _ = chunk_info(q)
                pltpu.make_async_copy(
                    words_hbm.at[row, pl.ds(c * _CHUNK, _CHUNK)],
                    idx_v.at[t],
                    semw,
                ).wait()
        for t in range(max_my):
            q = wid + _NWORK * t

            @pl.when(q < nc_total)
            def _():
                pltpu.async_copy(table_hbm.at[idx_v.at[t]], w_v.at[t], semw)
        for t in range(max_my):
            q = wid + _NWORK * t

            @pl.when(q < nc_total)
            def _():
                pltpu.make_async_copy(
                    table_hbm.at[idx_v.at[t]], w_v.at[t], semw
                ).wait()

        for t in range(max_my):
            q = wid + _NWORK * t

            @pl.when(q < nc_total)
            def _():
                _, c, lr = chunk_info(q)
                rem = lr - (lr // _CHUNK) * _CHUNK

                @pl.when((rem > 0) & (c == lr // _CHUNK))
                def _():
                    lanes = lax.iota(jnp.int32, _LANES)
                    for g in range(_CHUNK // _LANES):
                        pos = g * _LANES + lanes
                        wrow = w_v[t, pl.ds(g * _LANES, _LANES)]
                        w_v[t, pl.ds(g * _LANES, _LANES)] = jnp.where(
                            pos < rem, wrow, 0.0
                        )

        zvec = jnp.zeros((_LANES,), jnp.float32)

        def chunk_body(t, carry):
            q = wid + _NWORK * t
            buf = t % 2
            valid = q < nc_total

            @pl.when(wid + _NWORK * (t + 1) < nc_total)
            def _():
                rown, cn, _ = chunk_info(wid + _NWORK * (t + 1))
                pltpu.async_copy(
                    v_hbm.at[rown, pl.ds(cn * _CHUNK, _CHUNK)],
                    vbuf.at[(t + 1) % 2],
                    semv.at[(t + 1) % 2],
                )

            @pl.when(t > 0)
            def _():
                pltpu.make_async_copy(
                    pbuf, out_hbm.at[q - _NWORK], semo
                ).wait()

            @pl.when(valid)
            def _():
                row, c, lr = chunk_info(q)
                pltpu.make_async_copy(
                    v_hbm.at[row, pl.ds(c * _CHUNK, _CHUNK)],
                    vbuf.at[buf],
                    semv.at[buf],
                ).wait()
                rem = lr - (lr // _CHUNK) * _CHUNK

                @pl.when((rem > 0) & (c == lr // _CHUNK))
                def _():
                    def zrow(rowi, cz):
                        for l in range(nd):
                            vbuf[buf, rowi, pl.ds(l * _LANES, _LANES)] = zvec
                        return cz

                    lax.fori_loop(rem, _CHUNK, zrow, 0, unroll=False)

                zeros = tuple(
                    jnp.zeros((_LANES,), jnp.float32) for _ in range(3 * nd)
                )

                def group_body(g, acc_g):
                    wv = w_v[t, pl.ds(g * _LANES, _LANES)]
                    lanes = lax.iota(jnp.int32, _LANES)
                    accs = list(acc_g)
                    for k in range(_LANES):
                        wk = jnp.sum(jnp.where(lanes == k, wv, 0.0))
                        bw = jnp.broadcast_to(wk, (_LANES,))
                        tok = g * _LANES + k
                        for l in range(nd):
                            vt = vbuf[buf, tok, pl.ds(l * _LANES, _LANES)]
                            accs[l] = accs[l] + vt
                            accs[nd + l] = accs[nd + l] + jnp.abs(vt)
                            accs[2 * nd + l] = accs[2 * nd + l] + bw * vt
                    return tuple(accs)

                acc = lax.fori_loop(
                    0, _CHUNK // _LANES, group_body, zeros, unroll=False
                )
                for a in range(3):
                    for l in range(nd):
                        pbuf[a, pl.ds(l * _LANES, _LANES)] = acc[a * nd + l]

            @pl.when(jnp.logical_not(valid))
            def _():
                for a in range(3):
                    for l in range(nd):
                        pbuf[a, pl.ds(l * _LANES, _LANES)] = zvec

            pltpu.async_copy(pbuf, out_hbm.at[q], semo)
            return carry

        lax.fori_loop(0, max_my, chunk_body, 0, unroll=False)
        pltpu.make_async_copy(
            pbuf, out_hbm.at[wid + _NWORK * (max_my - 1)], semo
        ).wait()

    return main_kernel(v, slen, cums, words, table_pad)


def _tc_combine(partials, slen, cums):
    nslot, three_d = partials.shape
    b_dim = slen.shape[0]
    d_dim = three_d // 3

    def body(p_ref, slen_ref, cums_ref, y_ref, yh_ref):
        cum_incl = cums_ref[...]
        ncv = (slen_ref[...] + (_CHUNK - 1)) // _CHUNK
        cum_excl = cum_incl - ncv
        q = lax.broadcasted_iota(jnp.int32, (b_dim, nslot), 1)
        own = (cum_excl[:, None] <= q) & (q < cum_incl[:, None])
        s = jnp.dot(
            own.astype(jnp.float32),
            p_ref[...],
            preferred_element_type=jnp.float32,
        )
        y_ref[...] = s[:, :d_dim] / s[:, d_dim : 2 * d_dim]
        yh_ref[...] = s[:, 2 * d_dim :]

    return pl.pallas_call(
        body,
        out_shape=[
            jax.ShapeDtypeStruct((b_dim, d_dim), jnp.float32),
            jax.ShapeDtypeStruct((b_dim, d_dim), jnp.float32),
        ],
    )(partials, slen, cums)


def kernel(vector_sequence, sentence_length, word_sequence, W):
    b_dim, t_dim, d_dim = vector_sequence.shape
    vocab = W.shape[0]
    slen = sentence_length.astype(jnp.int32)
    words = word_sequence.astype(jnp.int32)
    cums = jnp.cumsum((slen + (_CHUNK - 1)) // _CHUNK).astype(jnp.int32)
    partials = _sc_main(vector_sequence, slen, cums, words, W)
    y, y_hat = _tc_combine(
        partials.reshape(_NSLOT, 3 * d_dim), slen, cums
    )
    return (y, y_hat)
